# baseline (speedup 1.0000x reference)
import functools

import jax
import jax.numpy as jnp
from jax import lax
from jax.experimental import pallas as pl
from jax.experimental.pallas import tpu as pltpu
from jax.experimental.pallas import tpu_sc as plsc

_LANE = 16
_NBUF = 2
_G = 2
_K = 128


def _cbow_sc(B, L, E, Vw):
    info = plsc.get_sparse_core_info()
    NC, NS = info.num_cores, info.num_subcores
    NW = NC * NS
    assert B % (NW * _G) == 0
    BPW = B // NW
    NCHUNK = BPW // _G
    EW = E // 2
    WC = EW // _LANE
    inv = 1.0 / L

    @functools.partial(
        pl.kernel,
        out_type=jax.ShapeDtypeStruct((B, E), jnp.float32),
        mesh=plsc.VectorSubcoreMesh(core_axis_name="c", subcore_axis_name="s"),
        compiler_params=pltpu.CompilerParams(use_tc_tiling_on_sc=False),
        scratch_types=[
            pltpu.VMEM((NCHUNK, _G * L), jnp.int32),
            pltpu.VMEM((_NBUF, _G * L, EW), jnp.int32),
            pltpu.VMEM((BPW, E), jnp.float32),
        ]
        + [pltpu.SemaphoreType.DMA] * _NBUF,
    )
    def cbow_kernel(data_hbm, w_hbm, out_hbm, idx_v, rows_v, out_v, *sems):
        wid = lax.axis_index("s") * NC + lax.axis_index("c")
        base = wid * NCHUNK
        pltpu.sync_copy(data_hbm.at[pl.ds(base, NCHUNK)], idx_v)
        for b in range(_NBUF):
            pltpu.async_copy(w_hbm.at[idx_v.at[b]], rows_v.at[b], sems[b])

        himask = jnp.full((_LANE,), -65536, jnp.int32)
        sh16 = jnp.full((_LANE,), 16, jnp.int32)

        def outer(g, carry):
            for b in range(_NBUF):
                k = g * _NBUF + b
                pltpu.make_async_copy(
                    w_hbm.at[idx_v.at[k]], rows_v.at[b], sems[b]
                ).wait()

                for e in range(_G):
                    def red(r, acc, b=b, e=e):
                        res = []
                        for c in range(WC):
                            u = rows_v[b, e * L + r, pl.ds(c * _LANE, _LANE)]
                            lo = lax.bitcast_convert_type(u << sh16, jnp.float32)
                            hi = lax.bitcast_convert_type(u & himask, jnp.float32)
                            res.append(acc[2 * c] + lo)
                            res.append(acc[2 * c + 1] + hi)
                        return tuple(res)

                    acc = lax.fori_loop(
                        0, L, red,
                        tuple(jnp.zeros((_LANE,), jnp.float32)
                              for _ in range(2 * WC)),
                    )
                    for c in range(WC):
                        out_v[k * _G + e, pl.ds((2 * c) * _LANE, _LANE)] = (
                            acc[2 * c] * inv)
                        out_v[k * _G + e, pl.ds((2 * c + 1) * _LANE, _LANE)] = (
                            acc[2 * c + 1] * inv)

                nk = k + _NBUF

                @pl.when(nk < NCHUNK)
                def _(b=b, nk=nk):
                    pltpu.async_copy(w_hbm.at[idx_v.at[nk]], rows_v.at[b], sems[b])

            return carry

        lax.fori_loop(0, NCHUNK // _NBUF, outer, 0)
        pltpu.sync_copy(out_v, out_hbm.at[pl.ds(wid * BPW, BPW)])

    return cbow_kernel


def _relayout_tc(V, E):
    grid = (V + 256 * _K - 1) // (256 * _K)

    def body(wt_ref, out_ref):
        for k in range(_K):
            x = wt_ref[:, pl.ds(256 * k, 256)]
            t = jnp.concatenate([x[:, 0:128].T, x[:, 128:256].T], axis=1)
            out_ref[pl.ds(128 * k, 128), :] = t.astype(jnp.bfloat16)

    rows_out = grid * 128 * _K
    return pl.pallas_call(
        body,
        grid=(grid,),
        in_specs=[pl.BlockSpec((E, 256 * _K), lambda i: (0, i))],
        out_specs=pl.BlockSpec((128 * _K, 2 * E), lambda i: (i, 0)),
        out_shape=jax.ShapeDtypeStruct((rows_out, 2 * E), jnp.bfloat16),
    )


def kernel(data, length, W):
    B, L = data.shape
    V, E = W.shape
    d = data.astype(jnp.int32)
    d = (d & ~jnp.int32(255)) | ((d & 127) << 1) | ((d >> 7) & 1)
    data_r = d.reshape(B // _G, _G * L)
    wbf = _relayout_tc(V, E)(W.T)
    rows = wbf.shape[0]
    w_i32 = lax.bitcast_convert_type(
        wbf.reshape(rows, E, 2), jnp.int32)
    w_words = w_i32.reshape(rows * 2, E // 2)
    out = _cbow_sc(B, L, E, w_words.shape[0])(data_r, w_words)
    return out.reshape(B, E // 32, 2, 16).transpose(0, 1, 3, 2).reshape(B, E)
